# Initial kernel scaffold; baseline (speedup 1.0000x reference)
import jax
import jax.numpy as jnp
from jax.experimental import pallas as pl


def kernel(obj_p0, obj_p1, obj_p2, delta_p0, delta_p1, delta_p2, anchors):
    raise NotImplementedError("write your pallas kernel here")


# speedup vs baseline: 16.9436x; 16.9436x over previous
import math

import jax
import jax.numpy as jnp
from jax.experimental import pallas as pl

_PRE = 1000
_POST = 1000
_NMS_THRESH = 0.7
_MIN_SIZE = 0.001
_IMG = 1216.0
_XFORM_CLIP = math.log(1000.0 / 16.0)
_NCAND = 3 * _PRE
_ROWS, _LANES = 24, 128
_NPAD = _ROWS * _LANES


def _nms_kernel(anc_ref, dlt_ref, lv_ref, box_ref, kept_ref):
    anc = anc_ref[0]
    dlt = dlt_ref[0]
    lv = lv_ref[0].astype(jnp.float32)

    ax1, ay1, ax2, ay2 = anc[0], anc[1], anc[2], anc[3]
    w = ax2 - ax1
    h = ay2 - ay1
    cx = ax1 + 0.5 * w
    cy = ay1 + 0.5 * h
    dx, dy = dlt[0], dlt[1]
    dw = jnp.minimum(dlt[2], _XFORM_CLIP)
    dh = jnp.minimum(dlt[3], _XFORM_CLIP)
    pcx = dx * w + cx
    pcy = dy * h + cy
    pw = jnp.exp(dw) * w
    ph = jnp.exp(dh) * h
    x1 = jnp.clip(pcx - 0.5 * pw, 0.0, _IMG)
    y1 = jnp.clip(pcy - 0.5 * ph, 0.0, _IMG)
    x2 = jnp.clip(pcx + 0.5 * pw, 0.0, _IMG)
    y2 = jnp.clip(pcy + 0.5 * ph, 0.0, _IMG)

    valid = ((x2 - x1) >= _MIN_SIZE) & ((y2 - y1) >= _MIN_SIZE) & (lv >= 0.0)

    neg = jnp.float32(-jnp.inf)
    coord_max = jnp.maximum(jnp.maximum(x1, x2), jnp.maximum(y1, y2))
    mc = jnp.max(jnp.where(valid, coord_max, neg))
    off = lv * (mc + 1.0)
    bx1 = x1 + off
    by1 = y1 + off
    bx2 = x2 + off
    by2 = y2 + off
    area = (bx2 - bx1) * (by2 - by1)

    fi = (jax.lax.broadcasted_iota(jnp.int32, (_ROWS, _LANES), 0) * _LANES
          + jax.lax.broadcasted_iota(jnp.int32, (_ROWS, _LANES), 1))

    def body(t, carry):
        supp, kept = carry
        oh = fi == t
        z = jnp.float32(0.0)
        ohf = jnp.where(oh, jnp.float32(1.0), z)
        is_keep = jnp.sum(supp * ohf) == 0.0
        bx1t = jnp.sum(jnp.where(oh, bx1, z))
        by1t = jnp.sum(jnp.where(oh, by1, z))
        bx2t = jnp.sum(jnp.where(oh, bx2, z))
        by2t = jnp.sum(jnp.where(oh, by2, z))
        at = jnp.sum(jnp.where(oh, area, z))
        iw = jnp.maximum(jnp.minimum(bx2, bx2t) - jnp.maximum(bx1, bx1t), 0.0)
        ih = jnp.maximum(jnp.minimum(by2, by2t) - jnp.maximum(by1, by1t), 0.0)
        inter = iw * ih
        iou = inter / jnp.maximum(area + at - inter, 1e-12)
        overlap = jnp.where(iou > _NMS_THRESH, jnp.float32(1.0), z)
        keepf = jnp.where(is_keep, jnp.float32(1.0), z)
        kept = jnp.maximum(kept, ohf * keepf)
        supp = jnp.maximum(supp, overlap * keepf)
        return supp, kept

    supp0 = jnp.where(valid, jnp.float32(0.0), jnp.float32(1.0))
    kept0 = jnp.zeros((_ROWS, _LANES), dtype=jnp.float32)
    _, kept = jax.lax.fori_loop(0, _NCAND, body, (supp0, kept0))

    box_ref[0, 0] = x1
    box_ref[0, 1] = y1
    box_ref[0, 2] = x2
    box_ref[0, 3] = y2
    kept_ref[0] = (kept > 0.0).astype(jnp.int32)


def kernel(obj_p0, obj_p1, obj_p2, delta_p0, delta_p1, delta_p2, anchors):
    N = obj_p0.shape[0]
    objs, dels = [], []
    for o, d in ((obj_p0, delta_p0), (obj_p1, delta_p1), (obj_p2, delta_p2)):
        _, A, H, W = o.shape
        objs.append(jnp.transpose(o, (0, 2, 3, 1)).reshape(N, -1))
        dels.append(
            jnp.transpose(d.reshape(N, A, 4, H, W), (0, 3, 4, 1, 2)).reshape(N, -1, 4))

    idx_all, sc_all = [], []
    offset = 0
    for ob in objs:
        s_, i_ = jax.lax.top_k(ob, _PRE)
        sc_all.append(s_)
        idx_all.append(i_ + offset)
        offset += ob.shape[1]
    idx = jnp.concatenate(idx_all, axis=1)
    scr = jnp.concatenate(sc_all, axis=1)
    lvl = jnp.concatenate(
        [jnp.full((_PRE,), i, jnp.int32) for i in range(3)])
    lvl = jnp.broadcast_to(lvl[None], (N, _NCAND))

    delta_cat = jnp.concatenate(dels, axis=1)
    bi = jnp.arange(N)[:, None]
    d_sel = delta_cat[bi, idx]
    a_sel = anchors[bi, idx]

    order = jnp.argsort(-scr, axis=1)
    lvl = jnp.take_along_axis(lvl, order, axis=1)
    d_sel = jnp.take_along_axis(d_sel, order[..., None], axis=1)
    a_sel = jnp.take_along_axis(a_sel, order[..., None], axis=1)

    pad = _NPAD - _NCAND
    lvl = jnp.pad(lvl, ((0, 0), (0, pad)), constant_values=-1)
    d_sel = jnp.pad(d_sel, ((0, 0), (0, pad), (0, 0)))
    a_sel = jnp.pad(a_sel, ((0, 0), (0, pad), (0, 0)))

    anc_t = jnp.transpose(a_sel, (0, 2, 1)).reshape(N, 4, _ROWS, _LANES)
    dlt_t = jnp.transpose(d_sel, (0, 2, 1)).reshape(N, 4, _ROWS, _LANES)
    lvl_r = lvl.reshape(N, _ROWS, _LANES)

    boxes_t, kept = pl.pallas_call(
        _nms_kernel,
        grid=(N,),
        in_specs=[
            pl.BlockSpec((1, 4, _ROWS, _LANES), lambda i: (i, 0, 0, 0)),
            pl.BlockSpec((1, 4, _ROWS, _LANES), lambda i: (i, 0, 0, 0)),
            pl.BlockSpec((1, _ROWS, _LANES), lambda i: (i, 0, 0)),
        ],
        out_specs=[
            pl.BlockSpec((1, 4, _ROWS, _LANES), lambda i: (i, 0, 0, 0)),
            pl.BlockSpec((1, _ROWS, _LANES), lambda i: (i, 0, 0)),
        ],
        out_shape=[
            jax.ShapeDtypeStruct((N, 4, _ROWS, _LANES), jnp.float32),
            jax.ShapeDtypeStruct((N, _ROWS, _LANES), jnp.int32),
        ],
    )(anc_t, dlt_t, lvl_r)

    boxes = boxes_t.reshape(N, 4, _NPAD).transpose(0, 2, 1)
    keptb = kept.reshape(N, _NPAD).astype(bool)

    rank = jnp.cumsum(keptb.astype(jnp.int32), axis=1) - 1
    pos = jnp.where(keptb & (rank < _POST), rank, _POST)
    outs = []
    for i in range(N):
        outs.append(
            jnp.zeros((_POST, 4), jnp.float32).at[pos[i]].set(
                boxes[i], mode='drop'))
    return jnp.stack(outs, axis=0)


# speedup vs baseline: 22.3113x; 1.3168x over previous
import math

import jax
import jax.numpy as jnp
from jax.experimental import pallas as pl

_PRE = 1000
_POST = 1000
_NMS_THRESH = 0.7
_MIN_SIZE = 0.001
_IMG = 1216.0
_XFORM_CLIP = math.log(1000.0 / 16.0)
_NCAND = 3 * _PRE
_ROWS, _LANES = 24, 128
_NPAD = _ROWS * _LANES


def _nms_kernel(anc_ref, dlt_ref, lv_ref, box_ref, kept_ref):
    anc = anc_ref[...]
    dlt = dlt_ref[...]
    lv = lv_ref[...].astype(jnp.float32)

    ax1, ay1, ax2, ay2 = anc[:, 0], anc[:, 1], anc[:, 2], anc[:, 3]
    w = ax2 - ax1
    h = ay2 - ay1
    cx = ax1 + 0.5 * w
    cy = ay1 + 0.5 * h
    dx, dy = dlt[:, 0], dlt[:, 1]
    dw = jnp.minimum(dlt[:, 2], _XFORM_CLIP)
    dh = jnp.minimum(dlt[:, 3], _XFORM_CLIP)
    pcx = dx * w + cx
    pcy = dy * h + cy
    pw = jnp.exp(dw) * w
    ph = jnp.exp(dh) * h
    x1 = jnp.clip(pcx - 0.5 * pw, 0.0, _IMG)
    y1 = jnp.clip(pcy - 0.5 * ph, 0.0, _IMG)
    x2 = jnp.clip(pcx + 0.5 * pw, 0.0, _IMG)
    y2 = jnp.clip(pcy + 0.5 * ph, 0.0, _IMG)

    valid = ((x2 - x1) >= _MIN_SIZE) & ((y2 - y1) >= _MIN_SIZE) & (lv >= 0.0)

    neg = jnp.float32(-jnp.inf)
    coord_max = jnp.maximum(jnp.maximum(x1, x2), jnp.maximum(y1, y2))
    mc = jnp.max(jnp.where(valid, coord_max, neg), axis=(1, 2), keepdims=True)
    off = lv * (mc + 1.0)
    bx1 = x1 + off
    by1 = y1 + off
    bx2 = x2 + off
    by2 = y2 + off
    area = (bx2 - bx1) * (by2 - by1)

    shape = x1.shape
    fi = (jax.lax.broadcasted_iota(jnp.int32, shape, 1) * _LANES
          + jax.lax.broadcasted_iota(jnp.int32, shape, 2))

    def body(t, carry):
        supp, kept = carry
        oh = fi == t
        z = jnp.float32(0.0)
        ohf = jnp.where(oh, jnp.float32(1.0), z)

        def ext(v):
            return jnp.sum(jnp.where(oh, v, z), axis=(1, 2), keepdims=True)

        is_keep = jnp.sum(supp * ohf, axis=(1, 2), keepdims=True) == 0.0
        bx1t = ext(bx1)
        by1t = ext(by1)
        bx2t = ext(bx2)
        by2t = ext(by2)
        at = ext(area)
        iw = jnp.maximum(jnp.minimum(bx2, bx2t) - jnp.maximum(bx1, bx1t), 0.0)
        ih = jnp.maximum(jnp.minimum(by2, by2t) - jnp.maximum(by1, by1t), 0.0)
        inter = iw * ih
        iou = inter / jnp.maximum(area + at - inter, 1e-12)
        overlap = jnp.where(iou > _NMS_THRESH, jnp.float32(1.0), z)
        keepf = jnp.where(is_keep, jnp.float32(1.0), z)
        kept = jnp.maximum(kept, ohf * keepf)
        supp = jnp.maximum(supp, overlap * keepf)
        return supp, kept

    supp0 = jnp.where(valid, jnp.float32(0.0), jnp.float32(1.0))
    kept0 = jnp.zeros(shape, dtype=jnp.float32)
    _, kept = jax.lax.fori_loop(0, _NCAND, body, (supp0, kept0))

    box_ref[:, 0] = x1
    box_ref[:, 1] = y1
    box_ref[:, 2] = x2
    box_ref[:, 3] = y2
    kept_ref[...] = (kept > 0.0).astype(jnp.int32)


def kernel(obj_p0, obj_p1, obj_p2, delta_p0, delta_p1, delta_p2, anchors):
    N = obj_p0.shape[0]
    objs, dels = [], []
    for o, d in ((obj_p0, delta_p0), (obj_p1, delta_p1), (obj_p2, delta_p2)):
        _, A, H, W = o.shape
        objs.append(jnp.transpose(o, (0, 2, 3, 1)).reshape(N, -1))
        dels.append(
            jnp.transpose(d.reshape(N, A, 4, H, W), (0, 3, 4, 1, 2)).reshape(N, -1, 4))

    idx_all, sc_all = [], []
    offset = 0
    for ob in objs:
        s_, i_ = jax.lax.top_k(ob, _PRE)
        sc_all.append(s_)
        idx_all.append(i_ + offset)
        offset += ob.shape[1]
    idx = jnp.concatenate(idx_all, axis=1)
    scr = jnp.concatenate(sc_all, axis=1)
    lvl = jnp.concatenate(
        [jnp.full((_PRE,), i, jnp.int32) for i in range(3)])
    lvl = jnp.broadcast_to(lvl[None], (N, _NCAND))

    delta_cat = jnp.concatenate(dels, axis=1)
    bi = jnp.arange(N)[:, None]
    d_sel = delta_cat[bi, idx]
    a_sel = anchors[bi, idx]

    order = jnp.argsort(-scr, axis=1)
    lvl = jnp.take_along_axis(lvl, order, axis=1)
    d_sel = jnp.take_along_axis(d_sel, order[..., None], axis=1)
    a_sel = jnp.take_along_axis(a_sel, order[..., None], axis=1)

    pad = _NPAD - _NCAND
    lvl = jnp.pad(lvl, ((0, 0), (0, pad)), constant_values=-1)
    d_sel = jnp.pad(d_sel, ((0, 0), (0, pad), (0, 0)))
    a_sel = jnp.pad(a_sel, ((0, 0), (0, pad), (0, 0)))

    anc_t = jnp.transpose(a_sel, (0, 2, 1)).reshape(N, 4, _ROWS, _LANES)
    dlt_t = jnp.transpose(d_sel, (0, 2, 1)).reshape(N, 4, _ROWS, _LANES)
    lvl_r = lvl.reshape(N, _ROWS, _LANES)

    boxes_t, kept = pl.pallas_call(
        _nms_kernel,
        out_shape=[
            jax.ShapeDtypeStruct((N, 4, _ROWS, _LANES), jnp.float32),
            jax.ShapeDtypeStruct((N, _ROWS, _LANES), jnp.int32),
        ],
    )(anc_t, dlt_t, lvl_r)

    boxes = boxes_t.reshape(N, 4, _NPAD).transpose(0, 2, 1)
    keptb = kept.reshape(N, _NPAD).astype(bool)

    rank = jnp.cumsum(keptb.astype(jnp.int32), axis=1) - 1
    pos = jnp.where(keptb & (rank < _POST), rank, _POST)
    outs = []
    for i in range(N):
        outs.append(
            jnp.zeros((_POST, 4), jnp.float32).at[pos[i]].set(
                boxes[i], mode='drop'))
    return jnp.stack(outs, axis=0)


# speedup vs baseline: 25.3658x; 1.1369x over previous
import math

import jax
import jax.numpy as jnp
from jax.experimental import pallas as pl

_PRE = 1000
_POST = 1000
_NMS_THRESH = 0.7
_MIN_SIZE = 0.001
_IMG = 1216.0
_XFORM_CLIP = math.log(1000.0 / 16.0)
_NCAND = 3 * _PRE
_ROWS, _LANES = 24, 128
_NPAD = _ROWS * _LANES


def _nms_kernel(anc_ref, dlt_ref, lv_ref, box_ref, kept_ref):
    anc = anc_ref[...]
    dlt = dlt_ref[...]
    lv = lv_ref[...].astype(jnp.float32)

    ax1, ay1, ax2, ay2 = anc[:, 0], anc[:, 1], anc[:, 2], anc[:, 3]
    w = ax2 - ax1
    h = ay2 - ay1
    cx = ax1 + 0.5 * w
    cy = ay1 + 0.5 * h
    dx, dy = dlt[:, 0], dlt[:, 1]
    dw = jnp.minimum(dlt[:, 2], _XFORM_CLIP)
    dh = jnp.minimum(dlt[:, 3], _XFORM_CLIP)
    pcx = dx * w + cx
    pcy = dy * h + cy
    pw = jnp.exp(dw) * w
    ph = jnp.exp(dh) * h
    x1 = jnp.clip(pcx - 0.5 * pw, 0.0, _IMG)
    y1 = jnp.clip(pcy - 0.5 * ph, 0.0, _IMG)
    x2 = jnp.clip(pcx + 0.5 * pw, 0.0, _IMG)
    y2 = jnp.clip(pcy + 0.5 * ph, 0.0, _IMG)

    valid = ((x2 - x1) >= _MIN_SIZE) & ((y2 - y1) >= _MIN_SIZE) & (lv >= 0.0)

    neg = jnp.float32(-jnp.inf)
    coord_max = jnp.maximum(jnp.maximum(x1, x2), jnp.maximum(y1, y2))
    mc = jnp.max(jnp.where(valid, coord_max, neg), axis=(1, 2), keepdims=True)
    off = lv * (mc + 1.0)
    bx1 = x1 + off
    by1 = y1 + off
    bx2 = x2 + off
    by2 = y2 + off
    area = (bx2 - bx1) * (by2 - by1)

    shape = x1.shape
    fi = (jax.lax.broadcasted_iota(jnp.int32, shape, 1) * _LANES
          + jax.lax.broadcasted_iota(jnp.int32, shape, 2))

    def cond(carry):
        t, _, _, cnt = carry
        return (t < _NCAND) & (jnp.min(cnt) < jnp.float32(_POST))

    def body(carry):
        t, supp, kept, cnt = carry
        oh = fi == t
        z = jnp.float32(0.0)
        ohf = jnp.where(oh, jnp.float32(1.0), z)

        def ext(v):
            return jnp.sum(jnp.where(oh, v, z), axis=(1, 2), keepdims=True)

        is_keep = jnp.sum(supp * ohf, axis=(1, 2), keepdims=True) == 0.0
        bx1t = ext(bx1)
        by1t = ext(by1)
        bx2t = ext(bx2)
        by2t = ext(by2)
        at = ext(area)
        iw = jnp.maximum(jnp.minimum(bx2, bx2t) - jnp.maximum(bx1, bx1t), 0.0)
        ih = jnp.maximum(jnp.minimum(by2, by2t) - jnp.maximum(by1, by1t), 0.0)
        inter = iw * ih
        iou = inter / jnp.maximum(area + at - inter, 1e-12)
        overlap = jnp.where(iou > _NMS_THRESH, jnp.float32(1.0), z)
        keepf = jnp.where(is_keep, jnp.float32(1.0), z)
        kept = jnp.maximum(kept, ohf * keepf)
        supp = jnp.maximum(supp, overlap * keepf)
        cnt = cnt + keepf
        return t + 1, supp, kept, cnt

    supp0 = jnp.where(valid, jnp.float32(0.0), jnp.float32(1.0))
    kept0 = jnp.zeros(shape, dtype=jnp.float32)
    cnt0 = jnp.zeros((shape[0], 1, 1), dtype=jnp.float32)
    _, _, kept, _ = jax.lax.while_loop(
        cond, body, (jnp.int32(0), supp0, kept0, cnt0))

    box_ref[:, 0] = x1
    box_ref[:, 1] = y1
    box_ref[:, 2] = x2
    box_ref[:, 3] = y2
    kept_ref[...] = (kept > 0.0).astype(jnp.int32)


def kernel(obj_p0, obj_p1, obj_p2, delta_p0, delta_p1, delta_p2, anchors):
    N = obj_p0.shape[0]
    objs, dels = [], []
    for o, d in ((obj_p0, delta_p0), (obj_p1, delta_p1), (obj_p2, delta_p2)):
        _, A, H, W = o.shape
        objs.append(jnp.transpose(o, (0, 2, 3, 1)).reshape(N, -1))
        dels.append(
            jnp.transpose(d.reshape(N, A, 4, H, W), (0, 3, 4, 1, 2)).reshape(N, -1, 4))

    idx_all, sc_all = [], []
    offset = 0
    for ob in objs:
        s_, i_ = jax.lax.top_k(ob, _PRE)
        sc_all.append(s_)
        idx_all.append(i_ + offset)
        offset += ob.shape[1]
    idx = jnp.concatenate(idx_all, axis=1)
    scr = jnp.concatenate(sc_all, axis=1)
    lvl = jnp.concatenate(
        [jnp.full((_PRE,), i, jnp.int32) for i in range(3)])
    lvl = jnp.broadcast_to(lvl[None], (N, _NCAND))

    delta_cat = jnp.concatenate(dels, axis=1)
    bi = jnp.arange(N)[:, None]
    d_sel = delta_cat[bi, idx]
    a_sel = anchors[bi, idx]

    order = jnp.argsort(-scr, axis=1)
    lvl = jnp.take_along_axis(lvl, order, axis=1)
    d_sel = jnp.take_along_axis(d_sel, order[..., None], axis=1)
    a_sel = jnp.take_along_axis(a_sel, order[..., None], axis=1)

    pad = _NPAD - _NCAND
    lvl = jnp.pad(lvl, ((0, 0), (0, pad)), constant_values=-1)
    d_sel = jnp.pad(d_sel, ((0, 0), (0, pad), (0, 0)))
    a_sel = jnp.pad(a_sel, ((0, 0), (0, pad), (0, 0)))

    anc_t = jnp.transpose(a_sel, (0, 2, 1)).reshape(N, 4, _ROWS, _LANES)
    dlt_t = jnp.transpose(d_sel, (0, 2, 1)).reshape(N, 4, _ROWS, _LANES)
    lvl_r = lvl.reshape(N, _ROWS, _LANES)

    boxes_t, kept = pl.pallas_call(
        _nms_kernel,
        out_shape=[
            jax.ShapeDtypeStruct((N, 4, _ROWS, _LANES), jnp.float32),
            jax.ShapeDtypeStruct((N, _ROWS, _LANES), jnp.int32),
        ],
    )(anc_t, dlt_t, lvl_r)

    boxes = boxes_t.reshape(N, 4, _NPAD).transpose(0, 2, 1)
    keptb = kept.reshape(N, _NPAD).astype(bool)

    rank = jnp.cumsum(keptb.astype(jnp.int32), axis=1) - 1
    pos = jnp.where(keptb & (rank < _POST), rank, _POST)
    outs = []
    for i in range(N):
        outs.append(
            jnp.zeros((_POST, 4), jnp.float32).at[pos[i]].set(
                boxes[i], mode='drop'))
    return jnp.stack(outs, axis=0)
